# baseline (speedup 1.0000x reference)
import jax
import jax.numpy as jnp
from jax import lax
from jax.experimental import pallas as pl
from jax.experimental.pallas import tpu as pltpu
from jax.experimental.pallas import tpu_sc as plsc

N = 10000
H = 128
E = 160000
K = 2
T = 4
G = 64
C = 104
NC, NS = 2, 16
NW = NC * NS
EW = K * E // NW
CH = 128
NCH = -(-EW // CH)
EWP = NCH * CH
JUNK = N
NPAD = N + 8
RPT = 624
VPAD = NW * 320
R = 1000
NB = N // R


def _mesh():
    return plsc.VectorSubcoreMesh(core_axis_name="c", subcore_axis_name="s",
                                  num_cores=NC, num_subcores=NS)


def _embed_body(emb_hbm, vid_hbm, out_hbm, vidx, vrows, sem):
    c = lax.axis_index("c")
    s = lax.axis_index("s")
    wid = s * NC + c
    base = wid * 320
    for j in range(5):
        b = base + j * 64
        pltpu.sync_copy(vid_hbm.at[pl.ds(b, 64)], vidx)
        pltpu.async_copy(emb_hbm.at[vidx], vrows, sem).wait()
        pltpu.sync_copy(vrows, out_hbm.at[pl.ds(b, 64)])


def _sc_embed(emb_table, vocab_ids_pad):
    return pl.kernel(
        _embed_body,
        out_type=jax.ShapeDtypeStruct((VPAD, H), jnp.float32),
        mesh=_mesh(),
        scratch_types=[
            pltpu.VMEM((64,), jnp.int32),
            pltpu.VMEM((64, H), jnp.float32),
            pltpu.SemaphoreType.DMA,
        ],
    )(emb_table, vocab_ids_pad)


def _msg_body(hw_hbm, src_hbm, dst_hbm, zeros_hbm, out_hbm,
              sidx_all, didx0, didx1, rows0, rows1, agg,
              sem0, sem1, dsem0, dsem1, zsem):
    c = lax.axis_index("c")
    s = lax.axis_index("s")
    wid = s * NC + c
    r0 = s * RPT

    @pl.when(s < NS - 1)
    def _():
        pltpu.async_copy(zeros_hbm.at[pl.ds(r0, RPT)],
                         agg.at[pl.ds(r0, RPT)], zsem)

    @pl.when(s == NS - 1)
    def _():
        lo = (NS - 1) * RPT
        pltpu.async_copy(zeros_hbm.at[pl.ds(lo, N - lo)],
                         agg.at[pl.ds(lo, N - lo)], zsem)

    pltpu.sync_copy(src_hbm.at[pl.ds(wid * EWP, EWP)], sidx_all)
    ebase = wid * EWP
    pltpu.async_copy(dst_hbm.at[pl.ds(ebase, CH)], didx0, dsem0)
    pltpu.async_copy(hw_hbm.at[sidx_all.at[pl.ds(0, CH)]], rows0, sem0)
    pltpu.async_copy(dst_hbm.at[pl.ds(ebase + CH, CH)], didx1, dsem1)
    pltpu.async_copy(hw_hbm.at[sidx_all.at[pl.ds(CH, CH)]], rows1, sem1)

    @pl.when(s < NS - 1)
    def _():
        pltpu.make_async_copy(zeros_hbm.at[pl.ds(r0, RPT)],
                              agg.at[pl.ds(r0, RPT)], zsem).wait()

    @pl.when(s == NS - 1)
    def _():
        lo = (NS - 1) * RPT
        pltpu.make_async_copy(zeros_hbm.at[pl.ds(lo, N - lo)],
                              agg.at[pl.ds(lo, N - lo)], zsem).wait()

    plsc.subcore_barrier()

    def pair(j2, carry):
        j0 = 2 * j2
        pltpu.make_async_copy(dst_hbm.at[pl.ds(ebase, CH)], didx0,
                              dsem0).wait()
        pltpu.make_async_copy(hw_hbm.at[sidx_all.at[pl.ds(0, CH)]], rows0,
                              sem0).wait()
        pltpu.sync_copy(rows0, agg.at[didx0], add=True)
        pltpu.async_copy(dst_hbm.at[pl.ds(ebase + (j0 + 2) * CH, CH)],
                         didx0, dsem0)
        pltpu.async_copy(hw_hbm.at[sidx_all.at[pl.ds((j0 + 2) * CH, CH)]],
                         rows0, sem0)
        pltpu.make_async_copy(dst_hbm.at[pl.ds(ebase, CH)], didx1,
                              dsem1).wait()
        pltpu.make_async_copy(hw_hbm.at[sidx_all.at[pl.ds(0, CH)]], rows1,
                              sem1).wait()
        pltpu.sync_copy(rows1, agg.at[didx1], add=True)

        @pl.when(j0 + 3 < NCH)
        def _():
            pltpu.async_copy(dst_hbm.at[pl.ds(ebase + (j0 + 3) * CH, CH)],
                             didx1, dsem1)
            pltpu.async_copy(hw_hbm.at[sidx_all.at[pl.ds((j0 + 3) * CH, CH)]],
                             rows1, sem1)

        return carry

    lax.fori_loop(0, (NCH - 1) // 2, pair, 0)

    pltpu.make_async_copy(dst_hbm.at[pl.ds(ebase, CH)], didx0, dsem0).wait()
    pltpu.make_async_copy(hw_hbm.at[sidx_all.at[pl.ds(0, CH)]], rows0,
                          sem0).wait()
    pltpu.sync_copy(rows0, agg.at[didx0], add=True)

    plsc.subcore_barrier()

    @pl.when(s < NS - 1)
    def _():
        pltpu.sync_copy(agg.at[pl.ds(r0, RPT)],
                        out_hbm.at[pl.ds(c * N + r0, RPT)])

    @pl.when(s == NS - 1)
    def _():
        lo = (NS - 1) * RPT
        pltpu.sync_copy(agg.at[pl.ds(lo, N - lo)],
                        out_hbm.at[pl.ds(c * N + lo, N - lo)])


def _sc_msg(hw, src, dst, zeros_n):
    return pl.kernel(
        _msg_body,
        out_type=jax.ShapeDtypeStruct((NC * N, H), jnp.float32),
        mesh=_mesh(),
        scratch_types=[
            pltpu.VMEM((EWP,), jnp.int32),
            pltpu.VMEM((CH,), jnp.int32),
            pltpu.VMEM((CH,), jnp.int32),
            pltpu.VMEM((CH, H), jnp.float32),
            pltpu.VMEM((CH, H), jnp.float32),
            pltpu.VMEM_SHARED((NPAD, H), jnp.float32),
            pltpu.SemaphoreType.DMA,
            pltpu.SemaphoreType.DMA,
            pltpu.SemaphoreType.DMA,
            pltpu.SemaphoreType.DMA,
            pltpu.SemaphoreType.DMA,
        ],
    )(hw, src, dst, zeros_n)


def _hw_body(h_ref, wm_ref, bm_ref, hw_ref):
    h = h_ref[...]
    w = wm_ref[...]
    b = bm_ref[...]
    hw_ref[0] = jnp.dot(h, w[0], preferred_element_type=jnp.float32) + b[0:1, :]
    hw_ref[1] = jnp.dot(h, w[1], preferred_element_type=jnp.float32) + b[1:2, :]


def _tc_hw(h, w_msg, b_msg):
    return pl.pallas_call(
        _hw_body,
        grid=(NB,),
        in_specs=[
            pl.BlockSpec((R, H), lambda i: (i, 0)),
            pl.BlockSpec((K, H, H), lambda i: (0, 0, 0)),
            pl.BlockSpec((K, H), lambda i: (0, 0)),
        ],
        out_specs=pl.BlockSpec((K, R, H), lambda i: (0, i, 0)),
        out_shape=jax.ShapeDtypeStruct((K, N, H), jnp.float32),
    )(h, w_msg, b_msg)


def _gru(aggp_ref, h_ref, wih_ref, whh_ref, bih_ref, bhh_ref):
    x = aggp_ref[0] + aggp_ref[1]
    h = h_ref[...]
    gi = jnp.dot(x, wih_ref[...], preferred_element_type=jnp.float32) + bih_ref[...]
    gh = jnp.dot(h, whh_ref[...], preferred_element_type=jnp.float32) + bhh_ref[...]
    r = jax.nn.sigmoid(gi[:, :H] + gh[:, :H])
    z = jax.nn.sigmoid(gi[:, H:2 * H] + gh[:, H:2 * H])
    n = jnp.tanh(gi[:, 2 * H:] + r * gh[:, 2 * H:])
    return (1.0 - z) * n + z * h


def _gru_hw_body(aggp_ref, h_ref, wih_ref, whh_ref, bih_ref, bhh_ref,
                 wm_ref, bm_ref, hnew_ref, hw_ref):
    hn = _gru(aggp_ref, h_ref, wih_ref, whh_ref, bih_ref, bhh_ref)
    hnew_ref[...] = hn
    w = wm_ref[...]
    b = bm_ref[...]
    hw_ref[0] = jnp.dot(hn, w[0], preferred_element_type=jnp.float32) + b[0:1, :]
    hw_ref[1] = jnp.dot(hn, w[1], preferred_element_type=jnp.float32) + b[1:2, :]


def _tc_gru_hw(aggp, h, w_ih, w_hh, b_ih, b_hh, w_msg, b_msg):
    return pl.pallas_call(
        _gru_hw_body,
        grid=(NB,),
        in_specs=[
            pl.BlockSpec((NC, R, H), lambda i: (0, i, 0)),
            pl.BlockSpec((R, H), lambda i: (i, 0)),
            pl.BlockSpec((H, 3 * H), lambda i: (0, 0)),
            pl.BlockSpec((H, 3 * H), lambda i: (0, 0)),
            pl.BlockSpec((1, 3 * H), lambda i: (0, 0)),
            pl.BlockSpec((1, 3 * H), lambda i: (0, 0)),
            pl.BlockSpec((K, H, H), lambda i: (0, 0, 0)),
            pl.BlockSpec((K, H), lambda i: (0, 0)),
        ],
        out_specs=[
            pl.BlockSpec((R, H), lambda i: (i, 0)),
            pl.BlockSpec((K, R, H), lambda i: (0, i, 0)),
        ],
        out_shape=[
            jax.ShapeDtypeStruct((N, H), jnp.float32),
            jax.ShapeDtypeStruct((K, N, H), jnp.float32),
        ],
    )(aggp, h, w_ih, w_hh, b_ih, b_hh, w_msg, b_msg)


def _final_body(aggp_ref, h_ref, raw_ref, wih_ref, whh_ref, bih_ref, bhh_ref,
                wga_ref, wgb_ref, bg_ref, wtr_ref, btr_ref, gid_ref, out_ref):
    hn = _gru(aggp_ref, h_ref, wih_ref, whh_ref, bih_ref, bhh_ref)
    gate = jax.nn.sigmoid(
        jnp.dot(hn, wga_ref[...], preferred_element_type=jnp.float32)
        + jnp.dot(raw_ref[...], wgb_ref[...], preferred_element_type=jnp.float32)
        + bg_ref[...])
    tr = jnp.dot(hn, wtr_ref[...], preferred_element_type=jnp.float32) + btr_ref[...]
    nodewise = gate * tr
    ids = gid_ref[0, 0, :].reshape(1, R)
    iota = lax.broadcasted_iota(jnp.int32, (G, R), 0)
    onehot = jnp.where(ids == iota, 1.0, 0.0).astype(jnp.float32)
    acc = jnp.dot(onehot, nodewise, preferred_element_type=jnp.float32)

    @pl.when(pl.program_id(0) == 0)
    def _():
        out_ref[...] = jnp.zeros_like(out_ref)

    out_ref[...] += acc


def _tc_final(aggp, h, raw_in, w_ih, w_hh, b_ih, b_hh,
              wga, wgb, bg, wtr, btr, gids3):
    return pl.pallas_call(
        _final_body,
        grid=(NB,),
        in_specs=[
            pl.BlockSpec((NC, R, H), lambda i: (0, i, 0)),
            pl.BlockSpec((R, H), lambda i: (i, 0)),
            pl.BlockSpec((R, H), lambda i: (i, 0)),
            pl.BlockSpec((H, 3 * H), lambda i: (0, 0)),
            pl.BlockSpec((H, 3 * H), lambda i: (0, 0)),
            pl.BlockSpec((1, 3 * H), lambda i: (0, 0)),
            pl.BlockSpec((1, 3 * H), lambda i: (0, 0)),
            pl.BlockSpec((H, H), lambda i: (0, 0)),
            pl.BlockSpec((H, H), lambda i: (0, 0)),
            pl.BlockSpec((1, H), lambda i: (0, 0)),
            pl.BlockSpec((H, H), lambda i: (0, 0)),
            pl.BlockSpec((1, H), lambda i: (0, 0)),
            pl.BlockSpec((1, 1, R), lambda i: (i, 0, 0)),
        ],
        out_specs=pl.BlockSpec((G, H), lambda i: (0, 0)),
        out_shape=jax.ShapeDtypeStruct((G, H), jnp.float32),
        compiler_params=pltpu.CompilerParams(
            dimension_semantics=("arbitrary",)),
    )(aggp, h, raw_in, w_ih, w_hh, b_ih, b_hh, wga, wgb, bg, wtr, btr, gids3)


def kernel(vocab_ids, labels, edge_lists, graph_nodes_list, num_graphs,
           emb_table, W_msg, b_msg, W_ih, W_hh, b_ih, b_hh,
           W_gate, b_gate, W_tr, b_tr):
    f32 = jnp.float32
    vocab_pad = jnp.pad(vocab_ids.astype(jnp.int32), (0, VPAD - N))
    src = (edge_lists[:, :, 0].astype(jnp.int32)
           + jnp.arange(K, dtype=jnp.int32)[:, None] * N).reshape(NW, EW)
    dst = edge_lists[:, :, 1].astype(jnp.int32).reshape(NW, EW)
    src = jnp.pad(src, ((0, 0), (0, EWP - EW))).reshape(-1)
    dst = jnp.pad(dst, ((0, 0), (0, EWP - EW)),
                  constant_values=JUNK).reshape(-1)
    zeros_n = jnp.zeros((N, H), f32)
    bmsg = b_msg.astype(f32)
    bih = b_ih.astype(f32).reshape(1, 3 * H)
    bhh = b_hh.astype(f32).reshape(1, 3 * H)
    wga = jnp.pad(W_gate[:H].astype(f32), ((0, 0), (0, H - C)))
    wgb = jnp.pad(W_gate[H:].astype(f32), ((0, 0), (0, H - C)))
    bg = jnp.pad(b_gate.astype(f32), (0, H - C)).reshape(1, H)
    wtr = jnp.pad(W_tr.astype(f32), ((0, 0), (0, H - C)))
    btr = jnp.pad(b_tr.astype(f32), (0, H - C)).reshape(1, H)
    gids3 = graph_nodes_list.astype(jnp.int32).reshape(NB, 1, R)

    raw_in = _sc_embed(emb_table.astype(f32), vocab_pad)[:N]
    hw = _tc_hw(raw_in, W_msg.astype(f32), bmsg).reshape(K * N, H)
    h = raw_in
    for t in range(T):
        aggp = _sc_msg(hw, src, dst, zeros_n).reshape(NC, N, H)
        if t < T - 1:
            h, hw3 = _tc_gru_hw(aggp, h, W_ih.astype(f32), W_hh.astype(f32),
                                bih, bhh, W_msg.astype(f32), bmsg)
            hw = hw3.reshape(K * N, H)
        else:
            out = _tc_final(aggp, h, raw_in, W_ih.astype(f32), W_hh.astype(f32),
                            bih, bhh, wga, wgb, bg, wtr, btr, gids3)
    return out[:, :C]


# speedup vs baseline: 1.2974x; 1.2974x over previous
import jax
import jax.numpy as jnp
from jax import lax
from jax.experimental import pallas as pl
from jax.experimental.pallas import tpu as pltpu
from jax.experimental.pallas import tpu_sc as plsc

N = 10000
H = 128
E = 160000
K = 2
T = 4
G = 64
C = 104
NC, NS = 2, 16
NW = NC * NS
EW = K * E // NW
CH = 128
NFULL = EW // CH
TAIL = EW - NFULL * CH
RPT = 624
VPAD = NW * 320
R = 1000
NB = N // R


def _mesh():
    return plsc.VectorSubcoreMesh(core_axis_name="c", subcore_axis_name="s",
                                  num_cores=NC, num_subcores=NS)


def _embed_body(emb_hbm, vid_hbm, out_hbm, vidx, vrows, sem):
    c = lax.axis_index("c")
    s = lax.axis_index("s")
    wid = s * NC + c
    base = wid * 320
    for j in range(5):
        b = base + j * 64
        pltpu.sync_copy(vid_hbm.at[pl.ds(b, 64)], vidx)
        pltpu.async_copy(emb_hbm.at[vidx], vrows, sem).wait()
        pltpu.sync_copy(vrows, out_hbm.at[pl.ds(b, 64)])


def _sc_embed(emb_table, vocab_ids_pad):
    return pl.kernel(
        _embed_body,
        out_type=jax.ShapeDtypeStruct((VPAD, H), jnp.float32),
        mesh=_mesh(),
        scratch_types=[
            pltpu.VMEM((64,), jnp.int32),
            pltpu.VMEM((64, H), jnp.float32),
            pltpu.SemaphoreType.DMA,
        ],
    )(emb_table, vocab_ids_pad)


def _msg_body(hw_hbm, src_hbm, dst_hbm, zeros_hbm, out_hbm,
              sidx_all, didx, didx_t, rows0, rows_t, agg, sem0, zsem):
    c = lax.axis_index("c")
    s = lax.axis_index("s")
    wid = s * NC + c
    r0 = s * RPT

    @pl.when(s < NS - 1)
    def _():
        pltpu.async_copy(zeros_hbm.at[pl.ds(r0, RPT)],
                         agg.at[pl.ds(r0, RPT)], zsem)

    @pl.when(s == NS - 1)
    def _():
        lo = (NS - 1) * RPT
        pltpu.async_copy(zeros_hbm.at[pl.ds(lo, N - lo)],
                         agg.at[pl.ds(lo, N - lo)], zsem)

    pltpu.sync_copy(src_hbm.at[pl.ds(wid * EW, EW)], sidx_all)

    @pl.when(s < NS - 1)
    def _():
        pltpu.make_async_copy(zeros_hbm.at[pl.ds(r0, RPT)],
                              agg.at[pl.ds(r0, RPT)], zsem).wait()

    @pl.when(s == NS - 1)
    def _():
        lo = (NS - 1) * RPT
        pltpu.make_async_copy(zeros_hbm.at[pl.ds(lo, N - lo)],
                              agg.at[pl.ds(lo, N - lo)], zsem).wait()

    plsc.subcore_barrier()

    ebase = wid * EW

    def chunk(j, carry):
        cpd = pltpu.async_copy(dst_hbm.at[pl.ds(ebase + j * CH, CH)],
                               didx, zsem)
        cpg = pltpu.async_copy(hw_hbm.at[sidx_all.at[pl.ds(j * CH, CH)]],
                               rows0, sem0)
        cpd.wait()
        cpg.wait()
        pltpu.sync_copy(rows0, agg.at[didx], add=True)
        return carry

    lax.fori_loop(0, NFULL, chunk, 0)

    pltpu.sync_copy(dst_hbm.at[pl.ds(ebase + NFULL * CH, TAIL)], didx_t)
    pltpu.async_copy(hw_hbm.at[sidx_all.at[pl.ds(NFULL * CH, TAIL)]],
                     rows_t, sem0).wait()
    pltpu.sync_copy(rows_t, agg.at[didx_t], add=True)

    plsc.subcore_barrier()

    @pl.when(s < NS - 1)
    def _():
        pltpu.sync_copy(agg.at[pl.ds(r0, RPT)],
                        out_hbm.at[pl.ds(c * N + r0, RPT)])

    @pl.when(s == NS - 1)
    def _():
        lo = (NS - 1) * RPT
        pltpu.sync_copy(agg.at[pl.ds(lo, N - lo)],
                        out_hbm.at[pl.ds(c * N + lo, N - lo)])


def _sc_msg(hw, src, dst, zeros_n):
    return pl.kernel(
        _msg_body,
        out_type=jax.ShapeDtypeStruct((NC * N, H), jnp.float32),
        mesh=_mesh(),
        scratch_types=[
            pltpu.VMEM((EW,), jnp.int32),
            pltpu.VMEM((CH,), jnp.int32),
            pltpu.VMEM((TAIL,), jnp.int32),
            pltpu.VMEM((CH, H), jnp.float32),
            pltpu.VMEM((TAIL, H), jnp.float32),
            pltpu.VMEM_SHARED((N, H), jnp.float32),
            pltpu.SemaphoreType.DMA,
            pltpu.SemaphoreType.DMA,
        ],
    )(hw, src, dst, zeros_n)


def _hw_body(h_ref, wm_ref, bm_ref, hw_ref):
    h = h_ref[...]
    w = wm_ref[...]
    b = bm_ref[...]
    hw_ref[0] = jnp.dot(h, w[0], preferred_element_type=jnp.float32) + b[0:1, :]
    hw_ref[1] = jnp.dot(h, w[1], preferred_element_type=jnp.float32) + b[1:2, :]


def _tc_hw(h, w_msg, b_msg):
    return pl.pallas_call(
        _hw_body,
        grid=(NB,),
        in_specs=[
            pl.BlockSpec((R, H), lambda i: (i, 0)),
            pl.BlockSpec((K, H, H), lambda i: (0, 0, 0)),
            pl.BlockSpec((K, H), lambda i: (0, 0)),
        ],
        out_specs=pl.BlockSpec((K, R, H), lambda i: (0, i, 0)),
        out_shape=jax.ShapeDtypeStruct((K, N, H), jnp.float32),
    )(h, w_msg, b_msg)


def _gru(aggp_ref, h_ref, wih_ref, whh_ref, bih_ref, bhh_ref):
    x = aggp_ref[0] + aggp_ref[1]
    h = h_ref[...]
    gi = jnp.dot(x, wih_ref[...], preferred_element_type=jnp.float32) + bih_ref[...]
    gh = jnp.dot(h, whh_ref[...], preferred_element_type=jnp.float32) + bhh_ref[...]
    r = jax.nn.sigmoid(gi[:, :H] + gh[:, :H])
    z = jax.nn.sigmoid(gi[:, H:2 * H] + gh[:, H:2 * H])
    n = jnp.tanh(gi[:, 2 * H:] + r * gh[:, 2 * H:])
    return (1.0 - z) * n + z * h


def _gru_hw_body(aggp_ref, h_ref, wih_ref, whh_ref, bih_ref, bhh_ref,
                 wm_ref, bm_ref, hnew_ref, hw_ref):
    hn = _gru(aggp_ref, h_ref, wih_ref, whh_ref, bih_ref, bhh_ref)
    hnew_ref[...] = hn
    w = wm_ref[...]
    b = bm_ref[...]
    hw_ref[0] = jnp.dot(hn, w[0], preferred_element_type=jnp.float32) + b[0:1, :]
    hw_ref[1] = jnp.dot(hn, w[1], preferred_element_type=jnp.float32) + b[1:2, :]


def _tc_gru_hw(aggp, h, w_ih, w_hh, b_ih, b_hh, w_msg, b_msg):
    return pl.pallas_call(
        _gru_hw_body,
        grid=(NB,),
        in_specs=[
            pl.BlockSpec((NC, R, H), lambda i: (0, i, 0)),
            pl.BlockSpec((R, H), lambda i: (i, 0)),
            pl.BlockSpec((H, 3 * H), lambda i: (0, 0)),
            pl.BlockSpec((H, 3 * H), lambda i: (0, 0)),
            pl.BlockSpec((1, 3 * H), lambda i: (0, 0)),
            pl.BlockSpec((1, 3 * H), lambda i: (0, 0)),
            pl.BlockSpec((K, H, H), lambda i: (0, 0, 0)),
            pl.BlockSpec((K, H), lambda i: (0, 0)),
        ],
        out_specs=[
            pl.BlockSpec((R, H), lambda i: (i, 0)),
            pl.BlockSpec((K, R, H), lambda i: (0, i, 0)),
        ],
        out_shape=[
            jax.ShapeDtypeStruct((N, H), jnp.float32),
            jax.ShapeDtypeStruct((K, N, H), jnp.float32),
        ],
    )(aggp, h, w_ih, w_hh, b_ih, b_hh, w_msg, b_msg)


def _final_body(aggp_ref, h_ref, raw_ref, wih_ref, whh_ref, bih_ref, bhh_ref,
                wga_ref, wgb_ref, bg_ref, wtr_ref, btr_ref, gid_ref, out_ref):
    hn = _gru(aggp_ref, h_ref, wih_ref, whh_ref, bih_ref, bhh_ref)
    gate = jax.nn.sigmoid(
        jnp.dot(hn, wga_ref[...], preferred_element_type=jnp.float32)
        + jnp.dot(raw_ref[...], wgb_ref[...], preferred_element_type=jnp.float32)
        + bg_ref[...])
    tr = jnp.dot(hn, wtr_ref[...], preferred_element_type=jnp.float32) + btr_ref[...]
    nodewise = gate * tr
    ids = gid_ref[0, 0, :].reshape(1, R)
    iota = lax.broadcasted_iota(jnp.int32, (G, R), 0)
    onehot = jnp.where(ids == iota, 1.0, 0.0).astype(jnp.float32)
    acc = jnp.dot(onehot, nodewise, preferred_element_type=jnp.float32)

    @pl.when(pl.program_id(0) == 0)
    def _():
        out_ref[...] = jnp.zeros_like(out_ref)

    out_ref[...] += acc


def _tc_final(aggp, h, raw_in, w_ih, w_hh, b_ih, b_hh,
              wga, wgb, bg, wtr, btr, gids3):
    return pl.pallas_call(
        _final_body,
        grid=(NB,),
        in_specs=[
            pl.BlockSpec((NC, R, H), lambda i: (0, i, 0)),
            pl.BlockSpec((R, H), lambda i: (i, 0)),
            pl.BlockSpec((R, H), lambda i: (i, 0)),
            pl.BlockSpec((H, 3 * H), lambda i: (0, 0)),
            pl.BlockSpec((H, 3 * H), lambda i: (0, 0)),
            pl.BlockSpec((1, 3 * H), lambda i: (0, 0)),
            pl.BlockSpec((1, 3 * H), lambda i: (0, 0)),
            pl.BlockSpec((H, H), lambda i: (0, 0)),
            pl.BlockSpec((H, H), lambda i: (0, 0)),
            pl.BlockSpec((1, H), lambda i: (0, 0)),
            pl.BlockSpec((H, H), lambda i: (0, 0)),
            pl.BlockSpec((1, H), lambda i: (0, 0)),
            pl.BlockSpec((1, 1, R), lambda i: (i, 0, 0)),
        ],
        out_specs=pl.BlockSpec((G, H), lambda i: (0, 0)),
        out_shape=jax.ShapeDtypeStruct((G, H), jnp.float32),
        compiler_params=pltpu.CompilerParams(
            dimension_semantics=("arbitrary",)),
    )(aggp, h, raw_in, w_ih, w_hh, b_ih, b_hh, wga, wgb, bg, wtr, btr, gids3)


def kernel(vocab_ids, labels, edge_lists, graph_nodes_list, num_graphs,
           emb_table, W_msg, b_msg, W_ih, W_hh, b_ih, b_hh,
           W_gate, b_gate, W_tr, b_tr):
    f32 = jnp.float32
    vocab_pad = jnp.pad(vocab_ids.astype(jnp.int32), (0, VPAD - N))
    src = (edge_lists[:, :, 0].astype(jnp.int32)
           + jnp.arange(K, dtype=jnp.int32)[:, None] * N).reshape(-1)
    dst = edge_lists[:, :, 1].astype(jnp.int32).reshape(-1)
    zeros_n = jnp.zeros((N, H), f32)
    bmsg = b_msg.astype(f32)
    bih = b_ih.astype(f32).reshape(1, 3 * H)
    bhh = b_hh.astype(f32).reshape(1, 3 * H)
    wga = jnp.pad(W_gate[:H].astype(f32), ((0, 0), (0, H - C)))
    wgb = jnp.pad(W_gate[H:].astype(f32), ((0, 0), (0, H - C)))
    bg = jnp.pad(b_gate.astype(f32), (0, H - C)).reshape(1, H)
    wtr = jnp.pad(W_tr.astype(f32), ((0, 0), (0, H - C)))
    btr = jnp.pad(b_tr.astype(f32), (0, H - C)).reshape(1, H)
    gids3 = graph_nodes_list.astype(jnp.int32).reshape(NB, 1, R)

    raw_in = _sc_embed(emb_table.astype(f32), vocab_pad)[:N]
    hw = _tc_hw(raw_in, W_msg.astype(f32), bmsg).reshape(K * N, H)
    h = raw_in
    for t in range(T):
        aggp = _sc_msg(hw, src, dst, zeros_n).reshape(NC, N, H)
        if t < T - 1:
            h, hw3 = _tc_gru_hw(aggp, h, W_ih.astype(f32), W_hh.astype(f32),
                                bih, bhh, W_msg.astype(f32), bmsg)
            hw = hw3.reshape(K * N, H)
        else:
            out = _tc_final(aggp, h, raw_in, W_ih.astype(f32), W_hh.astype(f32),
                            bih, bhh, wga, wgb, bg, wtr, btr, gids3)
    return out[:, :C]
